# Initial kernel scaffold; baseline (speedup 1.0000x reference)
import functools

import jax
import jax.numpy as jnp
from jax import lax
from jax.experimental import pallas as pl
from jax.experimental.pallas import tpu as pltpu
from jax.experimental.pallas import tpu_sc as plsc

_U = 25000
_N = 50000
_D = 64
_H = 32
_NSUB = 16
_NCORE = 2
_CHUNK = 128
_RPT = _N // _NSUB
_FCH = 125
_NFL = _RPT // _FCH


def _pad_edges(rows, cols, vals, ep):
    e = rows.shape[0]
    pad = ep - e
    if pad:
        rows = jnp.concatenate([rows, jnp.zeros((pad,), rows.dtype)])
        cols = jnp.concatenate([cols, jnp.zeros((pad,), cols.dtype)])
        vals = jnp.concatenate([vals, jnp.zeros((pad,), vals.dtype)])
    return rows, cols, vals


def _sc_spmm2(ego_lo, ego_hi, ra, ca, va, rb, cb, vb, ep):
    epw = ep // _NSUB
    nch = epw // _CHUNK
    mesh = plsc.VectorSubcoreMesh(core_axis_name="c", subcore_axis_name="s")

    @functools.partial(
        pl.kernel,
        mesh=mesh,
        out_type=(
            jax.ShapeDtypeStruct((_NCORE, _N, _H), jnp.float32),
            jax.ShapeDtypeStruct((_NCORE, _N, _H), jnp.float32),
        ),
        scratch_types=[
            pltpu.VMEM_SHARED((_N, _H), jnp.float32),
            pltpu.VMEM((_CHUNK,), jnp.int32),
            pltpu.VMEM((_CHUNK,), jnp.int32),
            pltpu.VMEM((_CHUNK,), jnp.float32),
            pltpu.VMEM((_CHUNK, _H), jnp.float32),
            pltpu.SemaphoreType.DMA,
        ],
    )
    def run(lo_hbm, hi_hbm, ra_h, ca_h, va_h, rb_h, cb_h, vb_h,
            outa_h, outb_h, acc, rows_v, cols_v, vals_v, gath, sem):
        c = lax.axis_index("c")
        s = lax.axis_index("s")
        my_r0 = s * _RPT

        def zero_acc():
            def zrow(i, carry):
                gath[i, pl.ds(0, 16)] = jnp.zeros((16,), jnp.float32)
                gath[i, pl.ds(16, 16)] = jnp.zeros((16,), jnp.float32)
                return carry
            lax.fori_loop(0, _FCH, zrow, 0)
            for k in range(_NFL):
                pltpu.sync_copy(gath.at[pl.ds(0, _FCH)],
                                acc.at[pl.ds(my_r0 + k * _FCH, _FCH)])

        def accumulate(tab_h, r_h, c_h, v_h):
            def chunk(g, carry):
                base = s * epw + g * _CHUNK
                pltpu.sync_copy(c_h.at[pl.ds(base, _CHUNK)], cols_v)
                pltpu.sync_copy(r_h.at[pl.ds(base, _CHUNK)], rows_v)
                pltpu.sync_copy(v_h.at[pl.ds(base, _CHUNK)], vals_v)
                pltpu.async_copy(tab_h.at[cols_v], gath, sem).wait()

                def srow(i, icarry):
                    vb16 = plsc.load_gather(
                        vals_v, [jnp.broadcast_to(i, (16,)).astype(jnp.int32)])
                    gath[i, pl.ds(0, 16)] = gath[i, pl.ds(0, 16)] * vb16
                    gath[i, pl.ds(16, 16)] = gath[i, pl.ds(16, 16)] * vb16
                    return icarry
                lax.fori_loop(0, _CHUNK, srow, 0)
                pltpu.sync_copy(gath, acc.at[rows_v], add=True)
                return carry
            lax.fori_loop(0, nch, chunk, 0)

        def flush(out_h):
            for k in range(_NFL):
                r0 = my_r0 + k * _FCH
                pltpu.sync_copy(acc.at[pl.ds(r0, _FCH)],
                                out_h.at[c, pl.ds(r0, _FCH)])

        def phase(tab_h, r_h, c_h, v_h, out_h):
            zero_acc()
            plsc.subcore_barrier()
            accumulate(tab_h, r_h, c_h, v_h)
            plsc.subcore_barrier()
            flush(out_h)
            plsc.subcore_barrier()

        @pl.when(c == 0)
        def _():
            phase(lo_hbm, ra_h, ca_h, va_h, outa_h)
            phase(lo_hbm, rb_h, cb_h, vb_h, outb_h)

        @pl.when(c == 1)
        def _():
            phase(hi_hbm, ra_h, ca_h, va_h, outa_h)
            phase(hi_hbm, rb_h, cb_h, vb_h, outb_h)

    return run(ego_lo, ego_hi, ra, ca, va, rb, cb, vb)


def kernel(user_emb, item_emb, edge_index, edge_vals,
           rand_edge_index, rand_edge_vals):
    ego = jnp.concatenate([user_emb, item_emb], axis=0)
    e = edge_vals.shape[0]
    step = _NSUB * _CHUNK
    ep = ((e + step - 1) // step) * step
    ra, ca, va = _pad_edges(edge_index[0], edge_index[1], edge_vals, ep)
    rb, cb, vb = _pad_edges(rand_edge_index[0], rand_edge_index[1],
                            rand_edge_vals, ep)
    a2, b2 = _sc_spmm2(ego[:, :_H], ego[:, _H:], ra, ca, va, rb, cb, vb, ep)
    agg = a2.transpose(1, 0, 2).reshape(_N, _D)
    rnd = b2.transpose(1, 0, 2).reshape(_N, _D)
    mean = 0.25 * ego + 0.75 * agg
    stack1 = jnp.stack([ego, agg, agg, agg], axis=1)
    stack2 = jnp.stack([ego, rnd, rnd, rnd], axis=1)
    return mean[:_U], mean[_U:], stack1, stack2


# speedup vs baseline: 2.6217x; 2.6217x over previous
import functools

import jax
import jax.numpy as jnp
from jax import lax
from jax.experimental import pallas as pl
from jax.experimental.pallas import tpu as pltpu
from jax.experimental.pallas import tpu_sc as plsc

_U = 25000
_N = 50000
_D = 64
_H = 32
_NSUB = 16
_NCORE = 2
_CHUNK = 128
_FCH = 400
_NFL = _N // _FCH


def _pad_edges(rows, cols, vals, ep):
    e = rows.shape[0]
    pad = ep - e
    if pad:
        rows = jnp.concatenate([rows, jnp.zeros((pad,), rows.dtype)])
        cols = jnp.concatenate([cols, jnp.zeros((pad,), cols.dtype)])
        vals = jnp.concatenate([vals, jnp.zeros((pad,), vals.dtype)])
    return rows, cols, vals


def _sc_spmm2(ego_lo, ego_hi, ra, ca, va, rb, cb, vb, ep):
    epw = ep // _NSUB
    nch = epw // _CHUNK
    mesh = plsc.VectorSubcoreMesh(core_axis_name="c", subcore_axis_name="s")

    @functools.partial(
        pl.kernel,
        mesh=mesh,
        compiler_params=pltpu.CompilerParams(use_tc_tiling_on_sc=False),
        out_type=(
            jax.ShapeDtypeStruct((_NCORE, _N, _H), jnp.float32),
            jax.ShapeDtypeStruct((_NCORE, _N, _H), jnp.float32),
        ),
        scratch_types=[
            pltpu.VMEM_SHARED((_N, _H), jnp.float32),
            pltpu.VMEM((_CHUNK,), jnp.int32),
            pltpu.VMEM((_CHUNK,), jnp.int32),
            pltpu.VMEM((_CHUNK,), jnp.float32),
            pltpu.VMEM((_CHUNK, _H), jnp.float32),
            pltpu.VMEM((_FCH, _H), jnp.float32),
            pltpu.SemaphoreType.DMA,
        ],
    )
    def run(lo_hbm, hi_hbm, ra_h, ca_h, va_h, rb_h, cb_h, vb_h,
            outa_h, outb_h, acc, rows_v, cols_v, vals_v, gath, zbuf, sem):
        c = lax.axis_index("c")
        s = lax.axis_index("s")

        def fill_zbuf():
            def zrow(i, carry):
                zbuf[i, pl.ds(0, 16)] = jnp.zeros((16,), jnp.float32)
                zbuf[i, pl.ds(16, 16)] = jnp.zeros((16,), jnp.float32)
                return carry
            lax.fori_loop(0, _FCH, zrow, 0)

        def each_owned_chunk(fn):
            def body(k, carry):
                idx = s + k * _NSUB
                @pl.when(idx < _NFL)
                def _():
                    fn(idx * _FCH)
                return carry
            lax.fori_loop(0, (_NFL + _NSUB - 1) // _NSUB, body, 0)

        def zero_acc():
            def zchunk(r0):
                pltpu.sync_copy(zbuf, acc.at[pl.ds(r0, _FCH)])
            each_owned_chunk(zchunk)

        def accumulate(tab_h, r_h, c_h, v_h):
            def chunk(g, carry):
                base = s * epw + g * _CHUNK
                pltpu.sync_copy(c_h.at[pl.ds(base, _CHUNK)], cols_v)
                pltpu.sync_copy(r_h.at[pl.ds(base, _CHUNK)], rows_v)
                pltpu.sync_copy(v_h.at[pl.ds(base, _CHUNK)], vals_v)
                pltpu.async_copy(tab_h.at[cols_v], gath, sem).wait()

                def sgroup(j16, icarry):
                    vv = vals_v[pl.ds(j16 * 16, 16)]
                    base_r = j16 * 16
                    for j in range(16):
                        vbj = jnp.broadcast_to(lax.slice(vv, (j,), (j + 1,)),
                                               (16,))
                        r = base_r + j
                        gath[r, pl.ds(0, 16)] = gath[r, pl.ds(0, 16)] * vbj
                        gath[r, pl.ds(16, 16)] = gath[r, pl.ds(16, 16)] * vbj
                    return icarry
                lax.fori_loop(0, _CHUNK // 16, sgroup, 0)
                pltpu.sync_copy(gath, acc.at[rows_v], add=True)
                return carry
            lax.fori_loop(0, nch, chunk, 0)

        def flush(out_h):
            def fchunk(r0):
                pltpu.sync_copy(acc.at[pl.ds(r0, _FCH)],
                                out_h.at[c, pl.ds(r0, _FCH)])
            each_owned_chunk(fchunk)

        def phase(tab_h, r_h, c_h, v_h, out_h):
            zero_acc()
            plsc.subcore_barrier()
            accumulate(tab_h, r_h, c_h, v_h)
            plsc.subcore_barrier()
            flush(out_h)
            plsc.subcore_barrier()

        fill_zbuf()

        @pl.when(c == 0)
        def _():
            phase(lo_hbm, ra_h, ca_h, va_h, outa_h)
            phase(lo_hbm, rb_h, cb_h, vb_h, outb_h)

        @pl.when(c == 1)
        def _():
            phase(hi_hbm, ra_h, ca_h, va_h, outa_h)
            phase(hi_hbm, rb_h, cb_h, vb_h, outb_h)

    return run(ego_lo, ego_hi, ra, ca, va, rb, cb, vb)


def kernel(user_emb, item_emb, edge_index, edge_vals,
           rand_edge_index, rand_edge_vals):
    ego = jnp.concatenate([user_emb, item_emb], axis=0)
    e = edge_vals.shape[0]
    step = _NSUB * _CHUNK
    ep = ((e + step - 1) // step) * step
    ra, ca, va = _pad_edges(edge_index[0], edge_index[1], edge_vals, ep)
    rb, cb, vb = _pad_edges(rand_edge_index[0], rand_edge_index[1],
                            rand_edge_vals, ep)
    a2, b2 = _sc_spmm2(ego[:, :_H], ego[:, _H:], ra, ca, va, rb, cb, vb, ep)
    agg = a2.transpose(1, 0, 2).reshape(_N, _D)
    rnd = b2.transpose(1, 0, 2).reshape(_N, _D)
    mean = 0.25 * ego + 0.75 * agg
    stack1 = jnp.stack([ego, agg, agg, agg], axis=1)
    stack2 = jnp.stack([ego, rnd, rnd, rnd], axis=1)
    return mean[:_U], mean[_U:], stack1, stack2
